# baseline (speedup 1.0000x reference)
import math

import jax
import jax.numpy as jnp
from jax.experimental import pallas as pl

NUM_NODES = 1000
NUM_PATCHES = 100
N = NUM_NODES * NUM_PATCHES
STATE_DIM = 10
HIDDEN = 128
MASK_RATIO = 0.25
B = 128
PRIOR_LOG = math.log(1.0 / N)


def _safe_log(x, eps=1e-08):
    return jnp.log(jnp.maximum(x, eps))


def _safe_normalize(x, eps=1e-08):
    denom = jnp.maximum(x.sum(axis=-1, keepdims=True), eps)
    return x / denom


def kernel(state, W1, b1, W2, b2, W3, b3, mask_history):
    h = jax.nn.gelu(state @ W1 + b1, approximate=False)
    h = jax.nn.gelu(h @ W2 + b2, approximate=False)
    logits = h @ W3 + b3
    probs = jax.nn.softmax(logits, axis=-1)
    mask_count = max(1, int(MASK_RATIO * N))
    g = jax.random.gumbel(jax.random.key(1), probs.shape, dtype=jnp.float32)
    perturbed = _safe_log(probs) + g
    _, indices = jax.lax.top_k(perturbed, mask_count)
    selected = jnp.maximum(jnp.take_along_axis(probs, indices, axis=1), 1e-08)
    log_prob = _safe_log(selected).sum(axis=1)
    batch_ar = jnp.arange(B)[:, None]
    mask = jnp.zeros_like(probs).at[batch_ar, indices].set(1.0)
    token_probs = _safe_normalize(mask)
    history = jnp.broadcast_to(mask_history[None, :], token_probs.shape)
    history = _safe_normalize(history)
    m = 0.5 * (token_probs + history)
    jsd_part1 = jnp.sum(token_probs * (_safe_log(token_probs) - _safe_log(m)), axis=-1)
    jsd_part2 = jnp.sum(history * (_safe_log(history) - _safe_log(m)), axis=-1)
    jsd_batch = 0.5 * (jsd_part1 + jsd_part2)
    kl_batch = jnp.sum(probs * (_safe_log(probs) - PRIOR_LOG), axis=-1)
    return (mask.reshape(B, NUM_NODES, NUM_PATCHES), indices, log_prob, probs, jsd_batch, kl_batch, token_probs)


# speedup vs baseline: 1.7918x; 1.7918x over previous
import functools
import math

import jax
import jax.numpy as jnp
from jax import lax
from jax.experimental import pallas as pl
from jax.experimental.pallas import tpu as pltpu
from jax.experimental.pallas import tpu_sc as plsc

_NUM_NODES = 1000
_NUM_PATCHES = 100
_N = _NUM_NODES * _NUM_PATCHES
_B = 128
_K = 25000
_PRIOR_LOG = math.log(1.0 / _N)
_BLK = 2048
_NBLK = (_N + _BLK - 1) // _BLK
_NEG = jnp.float32(-3.0e38)


def _safelog(x):
    return jnp.log(jnp.maximum(x, 1e-8))


def _max_kernel(h2_ref, w3_ref, b3_ref, m_ref):
    j = pl.program_id(0)
    l = jnp.dot(h2_ref[...], w3_ref[...], preferred_element_type=jnp.float32)
    l = l + b3_ref[...]
    col = jax.lax.broadcasted_iota(jnp.int32, (_B, _BLK), 1) + j * _BLK
    lm = jnp.where(col < _N, l, -jnp.inf)
    bm = jnp.max(lm, axis=1, keepdims=True)

    @pl.when(j == 0)
    def _():
        m_ref[...] = bm

    @pl.when(j > 0)
    def _():
        m_ref[...] = jnp.maximum(m_ref[...], bm)


_WBLK = 7696
_NWIN = 13


def _ut_kernel(h2_ref, w3_ref, b3_ref, m_ref, ut_ref):
    j = pl.program_id(0)
    l = jnp.dot(h2_ref[...], w3_ref[...], preferred_element_type=jnp.float32)
    l = l + b3_ref[...]
    u = jnp.exp(l - m_ref[...])
    col = jax.lax.broadcasted_iota(jnp.int32, (_B, _BLK), 1) + j * _BLK
    u = jnp.where(col < _N, u, 0.0)
    ut_ref[...] = jnp.transpose(u)


def _sum_kernel(ut_ref, s_ref):
    def body(k, acc):
        return acc + ut_ref[pl.ds(k * 8, 8), :]

    acc = jax.lax.fori_loop(0, _WBLK // 8, body, jnp.zeros((8, _B), jnp.float32))
    r = acc[0:4] + acc[4:8]
    r = r[0:2] + r[2:4]
    r = r[0:1] + r[1:2]

    j = pl.program_id(0)

    @pl.when(j == 0)
    def _():
        s_ref[...] = jnp.zeros((1, _B), jnp.float32)

    s_ref[...] = s_ref[...] + r


def _probs_kernel(h2_ref, w3_ref, b3_ref, g_ref, m_ref, s_ref,
                  p_ref, v_ref, kl_ref):
    j = pl.program_id(0)
    l = jnp.dot(h2_ref[...], w3_ref[...], preferred_element_type=jnp.float32)
    l = l + b3_ref[...]
    u = jnp.exp(l - m_ref[...])
    p = u / s_ref[...]
    p_ref[...] = p
    lp = _safelog(p)
    v_ref[...] = lp + g_ref[...]
    col = jax.lax.broadcasted_iota(jnp.int32, (_B, _BLK), 1) + j * _BLK
    klt = jnp.where(col < _N, p * (lp - _PRIOR_LOG), 0.0)

    @pl.when(j == 0)
    def _():
        kl_ref[...] = jnp.zeros((_B, 1), jnp.float32)

    kl_ref[...] = kl_ref[...] + jnp.sum(klt, axis=1, keepdims=True)


def _a_chain(h2, W3, b3, g):
    b3r = b3.reshape(1, _N)
    m = pl.pallas_call(
        _max_kernel,
        grid=(_NBLK,),
        in_specs=[
            pl.BlockSpec((_B, 128), lambda j: (0, 0)),
            pl.BlockSpec((128, _BLK), lambda j: (0, j)),
            pl.BlockSpec((1, _BLK), lambda j: (0, j)),
        ],
        out_specs=pl.BlockSpec((_B, 1), lambda j: (0, 0)),
        out_shape=jax.ShapeDtypeStruct((_B, 1), jnp.float32),
    )(h2, W3, b3r)
    ut = pl.pallas_call(
        _ut_kernel,
        grid=(_NBLK,),
        in_specs=[
            pl.BlockSpec((_B, 128), lambda j: (0, 0)),
            pl.BlockSpec((128, _BLK), lambda j: (0, j)),
            pl.BlockSpec((1, _BLK), lambda j: (0, j)),
            pl.BlockSpec((_B, 1), lambda j: (0, 0)),
        ],
        out_specs=pl.BlockSpec((_BLK, _B), lambda j: (j, 0)),
        out_shape=jax.ShapeDtypeStruct((_NWIN * _WBLK, _B), jnp.float32),
    )(h2, W3, b3r, m)
    st = pl.pallas_call(
        _sum_kernel,
        grid=(_NWIN,),
        in_specs=[pl.BlockSpec((_WBLK, _B), lambda j: (j, 0))],
        out_specs=pl.BlockSpec((1, _B), lambda j: (0, 0)),
        out_shape=jax.ShapeDtypeStruct((1, _B), jnp.float32),
    )(ut)
    s = st.reshape(_B, 1)
    probs, pert, kl = pl.pallas_call(
        _probs_kernel,
        grid=(_NBLK,),
        in_specs=[
            pl.BlockSpec((_B, 128), lambda j: (0, 0)),
            pl.BlockSpec((128, _BLK), lambda j: (0, j)),
            pl.BlockSpec((1, _BLK), lambda j: (0, j)),
            pl.BlockSpec((_B, _BLK), lambda j: (0, j)),
            pl.BlockSpec((_B, 1), lambda j: (0, 0)),
            pl.BlockSpec((_B, 1), lambda j: (0, 0)),
        ],
        out_specs=[
            pl.BlockSpec((_B, _BLK), lambda j: (0, j)),
            pl.BlockSpec((_B, _BLK), lambda j: (0, j)),
            pl.BlockSpec((_B, 1), lambda j: (0, 0)),
        ],
        out_shape=[
            jax.ShapeDtypeStruct((_B, _N), jnp.float32),
            jax.ShapeDtypeStruct((_B, _N), jnp.float32),
            jax.ShapeDtypeStruct((_B, 1), jnp.float32),
        ],
    )(h2, W3, b3r, g, m, s)
    return probs, pert, kl[:, 0]


_TB = 8


def _skey(v):
    b = lax.bitcast_convert_type(v, jnp.int32)
    return jnp.where(b >= 0, b, b ^ jnp.int32(0x7FFFFFFF))


def _thresh_kernel(v_ref, t_ref):
    skey = _skey(v_ref[...])

    def body(i, tu):
        bit = jnp.int32(1) << (31 - i)
        cand = tu | bit
        cand_s = cand ^ jnp.int32(-2147483648)
        cnt = jnp.sum((skey >= cand_s).astype(jnp.int32), axis=1, keepdims=True)
        return jnp.where(cnt >= _K, cand, tu)

    tu = lax.fori_loop(0, 32, body, jnp.zeros((_TB, 1), jnp.int32))
    t_ref[...] = tu ^ jnp.int32(-2147483648)


def _threshold(pert):
    return pl.pallas_call(
        _thresh_kernel,
        grid=(_B // _TB,),
        in_specs=[pl.BlockSpec((_TB, _N), lambda j: (j, 0))],
        out_specs=pl.BlockSpec((_TB, 1), lambda j: (j, 0)),
        out_shape=jax.ShapeDtypeStruct((_B, 1), jnp.int32),
    )(pert)


_CAND = 32768
_NSC, _NTEC = 2, 16
_NW = _NSC * _NTEC
_RPW = _B // _NW
_HALF = _N // 2
_SENT = jnp.float32(-3.0e38)

def _compact_body(v_hbm, t_hbm, oval_hbm, oidx_hbm, vbuf, valbuf, idxbuf, tbuf):
    wid = lax.axis_index("s") * _NSC + lax.axis_index("c")
    lanes = lax.iota(jnp.int32, 16)

    for k in range(_RPW):
        r = wid * _RPW + k
        pltpu.sync_copy(t_hbm.at[pl.ds(r * 16, 16)], tbuf)
        tvec = tbuf[pl.ds(0, 16)]

        def fill(i, _):
            valbuf[pl.ds(i * 16, 16)] = jnp.full((16,), _SENT, jnp.float32)
            idxbuf[pl.ds(i * 16, 16)] = jnp.zeros((16,), jnp.int32)
            return 0

        lax.fori_loop(0, _CAND // 16, fill, 0)

        off = jnp.int32(0)
        for h in range(2):
            pltpu.sync_copy(v_hbm.at[pl.ds(r * _N + h * _HALF, _HALF)], vbuf)

            def body(i, off):
                v16 = vbuf[pl.ds(i * 16, 16)]
                key = _skey(v16)
                m = key >= tvec
                mi = m.astype(jnp.int32)
                cs = plsc.cumsum(mi)
                o = jnp.minimum(off, _CAND - 16)
                dest = jnp.where(m, o + cs - 1, _CAND + lanes)
                plsc.store_scatter(valbuf, [dest], v16)
                idx16 = lanes + (h * _HALF + i * 16)
                plsc.store_scatter(idxbuf, [dest], idx16)
                return off + jnp.sum(mi, axis=0)

            off = lax.fori_loop(0, _HALF // 16, body, off)

        pltpu.sync_copy(valbuf.at[pl.ds(0, _CAND)],
                        oval_hbm.at[pl.ds(r * _CAND, _CAND)])
        pltpu.sync_copy(idxbuf.at[pl.ds(0, _CAND)],
                        oidx_hbm.at[pl.ds(r * _CAND, _CAND)])


@functools.cache
def _get_compact():
    mesh = plsc.VectorSubcoreMesh(core_axis_name="c", subcore_axis_name="s")
    return pl.kernel(
        _compact_body,
        out_type=[
            jax.ShapeDtypeStruct((_B * _CAND,), jnp.float32),
            jax.ShapeDtypeStruct((_B * _CAND,), jnp.int32),
        ],
        mesh=mesh,
        scratch_types=[
            pltpu.VMEM((_HALF,), jnp.float32),
            pltpu.VMEM((_CAND + 16,), jnp.float32),
            pltpu.VMEM((_CAND + 16,), jnp.int32),
            pltpu.VMEM((16,), jnp.int32),
        ],
    )


_R = 1024
_C = 128
_NSORT = _R * _C


def _rollc(x, s, axis):
    n = x.shape[axis]
    s = s % n
    if s == 0:
        return x
    if axis == 0:
        return jnp.concatenate([x[n - s:], x[:n - s]], 0)
    return jnp.concatenate([x[:, n - s:], x[:, :n - s]], 1)


def _sort_kernel(v_ref, o_ref, ov_ref):
    v = jnp.transpose(v_ref[0])
    r_io = lax.broadcasted_iota(jnp.int32, (_R, _C), 0)
    c_io = lax.broadcasted_iota(jnp.int32, (_R, _C), 1)
    ii = c_io * _R + r_io

    def pbit(d):
        if d < _R:
            return (r_io & d) != 0
        return (c_io & (d // _R)) != 0

    def partner(x, d):
        if d < _R:
            bit = pbit(d)
            return jnp.where(bit, _rollc(x, d, 0), _rollc(x, -d, 0))
        dl = d // _R
        bit = pbit(d)
        return jnp.where(bit, _rollc(x, dl, 1), _rollc(x, -dl, 1))

    bs = 2
    while bs <= _NSORT:
        d = bs // 2
        while d >= 1:
            pv = partner(v, d)
            pi = partner(ii, d)
            first = (v > pv) | ((v == pv) & (ii < pi))
            is_upper = pbit(d)
            if bs < _NSORT:
                dirdesc = jnp.logical_not(pbit(bs))
                keep = (first ^ dirdesc) == is_upper
            else:
                keep = first ^ is_upper
            v = jnp.where(keep, v, pv)
            ii = jnp.where(keep, ii, pi)
            d //= 2
        bs *= 2
    o_ref[0] = jnp.transpose(ii)
    ov_ref[0] = jnp.transpose(v)


def _sort(cval3):
    return pl.pallas_call(
        _sort_kernel,
        grid=(_B,),
        in_specs=[
            pl.BlockSpec((1, _C, _R), lambda b: (b, 0, 0)),
        ],
        out_specs=[
            pl.BlockSpec((1, _C, _R), lambda b: (b, 0, 0)),
            pl.BlockSpec((1, _C, _R), lambda b: (b, 0, 0)),
        ],
        out_shape=[
            jax.ShapeDtypeStruct((_B, _C, _R), jnp.int32),
            jax.ShapeDtypeStruct((_B, _C, _R), jnp.float32),
        ],
    )(cval3)


_IDXPAD = ((_K + 15) // 16) * 16


def _scatter_body(idx_hbm, mask_hbm, rowbuf, idxbuf):
    wid = lax.axis_index("s") * _NSC + lax.axis_index("c")
    lanes = lax.iota(jnp.int32, 16)
    ones = jnp.ones((16,), jnp.float32)

    for k in range(_RPW):
        r = wid * _RPW + k

        def zero(i, _):
            rowbuf[pl.ds(i * 16, 16)] = jnp.zeros((16,), jnp.float32)
            return 0

        lax.fori_loop(0, _N // 16, zero, 0)
        pltpu.sync_copy(idx_hbm.at[pl.ds(r * _CAND, _IDXPAD)], idxbuf)

        def body(i, _):
            i16 = idxbuf[pl.ds(i * 16, 16)]
            plsc.store_scatter(rowbuf, [i16], ones)
            return 0

        lax.fori_loop(0, _K // 16, body, 0)
        i16 = idxbuf[pl.ds((_K // 16) * 16, 16)]
        dest = jnp.where(lanes < (_K % 16), i16, _N + lanes)
        plsc.store_scatter(rowbuf, [dest], ones)
        pltpu.sync_copy(rowbuf.at[pl.ds(0, _N)], mask_hbm.at[pl.ds(r * _N, _N)])


@functools.cache
def _get_scatter():
    mesh = plsc.VectorSubcoreMesh(core_axis_name="c", subcore_axis_name="s")
    return pl.kernel(
        _scatter_body,
        out_type=jax.ShapeDtypeStruct((_B * _N,), jnp.float32),
        mesh=mesh,
        scratch_types=[
            pltpu.VMEM((_N + 16,), jnp.float32),
            pltpu.VMEM((_IDXPAD,), jnp.int32),
        ],
    )


def _final_kernel(v_ref, p_ref, mhb_ref, vk_ref, ik_ref,
                  msk_ref, tok_ref, lp_ref, jsd_ref):
    j = pl.program_id(0)
    v = v_ref[...]
    p = p_ref[...]
    vk = vk_ref[...]
    ik = ik_ref[...]
    col = lax.broadcasted_iota(jnp.int32, (_B, _BLK), 1) + j * _BLK
    valid = col < _N
    msk = jnp.where((v > vk) | ((v == vk) & (col <= ik)), 1.0, 0.0)
    msk = jnp.where(valid, msk, 0.0)
    msk_ref[...] = msk
    t = msk / jnp.float32(_K)
    tok_ref[...] = t
    h = mhb_ref[...]
    mm = 0.5 * (t + h)
    jterm = (t * (_safelog(t) - _safelog(mm))
             + h * (_safelog(h) - _safelog(mm)))
    jterm = jnp.where(valid, jterm, 0.0)
    lp_t = jnp.where(valid, msk * _safelog(p), 0.0)

    @pl.when(j == 0)
    def _():
        lp_ref[...] = jnp.zeros((_B, 1), jnp.float32)
        jsd_ref[...] = jnp.zeros((_B, 1), jnp.float32)

    lp_ref[...] = lp_ref[...] + jnp.sum(lp_t, axis=1, keepdims=True)
    jsd_ref[...] = jsd_ref[...] + 0.5 * jnp.sum(jterm, axis=1, keepdims=True)


def _final(perturbed, probs, hnorm, vk, ik):
    return pl.pallas_call(
        _final_kernel,
        grid=(_NBLK,),
        in_specs=[
            pl.BlockSpec((_B, _BLK), lambda j: (0, j)),
            pl.BlockSpec((_B, _BLK), lambda j: (0, j)),
            pl.BlockSpec((1, _BLK), lambda j: (0, j)),
            pl.BlockSpec((_B, 1), lambda j: (0, 0)),
            pl.BlockSpec((_B, 1), lambda j: (0, 0)),
        ],
        out_specs=[
            pl.BlockSpec((_B, _BLK), lambda j: (0, j)),
            pl.BlockSpec((_B, _BLK), lambda j: (0, j)),
            pl.BlockSpec((_B, 1), lambda j: (0, 0)),
            pl.BlockSpec((_B, 1), lambda j: (0, 0)),
        ],
        out_shape=[
            jax.ShapeDtypeStruct((_B, _N), jnp.float32),
            jax.ShapeDtypeStruct((_B, _N), jnp.float32),
            jax.ShapeDtypeStruct((_B, 1), jnp.float32),
            jax.ShapeDtypeStruct((_B, 1), jnp.float32),
        ],
    )(perturbed, probs, hnorm, vk, ik)


def kernel(state, W1, b1, W2, b2, W3, b3, mask_history):
    h = jax.nn.gelu(state @ W1 + b1, approximate=False)
    h2 = jax.nn.gelu(h @ W2 + b2, approximate=False)
    g = jax.random.gumbel(jax.random.key(1), (_B, _N), dtype=jnp.float32)

    probs, perturbed, kl_batch = _a_chain(h2, W3, b3, g)

    vpad = jnp.concatenate(
        [perturbed, jnp.full((_B, _NSORT - _N), _SENT, jnp.float32)], axis=1)
    sidx, sval = _sort(vpad.reshape(_B, _C, _R))
    idx_full = sidx.reshape(_B, _NSORT)
    indices = idx_full[:, :_K]
    vk = sval.reshape(_B, _NSORT)[:, _K - 1:_K]
    ik = idx_full[:, _K - 1:_K]

    mh = mask_history.reshape(1, _N)
    hnorm = mh / jnp.maximum(mh.sum(axis=-1, keepdims=True), 1e-8)

    mask_flat, token_probs, lp, jsd = _final(perturbed, probs, hnorm, vk, ik)
    return (mask_flat.reshape(_B, _NUM_NODES, _NUM_PATCHES), indices,
            lp[:, 0], probs, jsd[:, 0], kl_batch, token_probs)
